# Initial kernel scaffold; baseline (speedup 1.0000x reference)
import jax
import jax.numpy as jnp
from jax.experimental import pallas as pl


def kernel(adj, input_emb, seq_lens, W0, b0, W1, b1, ln_gamma, ln_beta):
    raise NotImplementedError("write your pallas kernel here")


# speedup vs baseline: 1.7926x; 1.7926x over previous
import jax
import jax.numpy as jnp
from jax.experimental import pallas as pl


def _gcn_fused_kernel(adj_ref, x_ref, w0_ref, b0_ref, w1_ref, b1_ref,
                      g_ref, beta_ref, out_ref, rs_ref, cs_ref):
    a = adj_ref[0]
    x = x_ref[0]
    rs = jnp.sum(a, axis=1, keepdims=True)
    cs = jnp.sum(a, axis=0, keepdims=True)
    denom = rs + 1.0

    h = jnp.dot(a, x, preferred_element_type=jnp.float32) + x
    h = jnp.dot(h, w0_ref[...], preferred_element_type=jnp.float32) + 2.0 * b0_ref[...]
    h = jax.nn.relu(h / denom)

    h2 = jnp.dot(a, h, preferred_element_type=jnp.float32) + h
    h2 = jnp.dot(h2, w1_ref[...], preferred_element_type=jnp.float32) + 2.0 * b1_ref[...]
    h2 = jax.nn.relu(h2 / denom)

    mu = jnp.mean(h2, axis=-1, keepdims=True)
    var = jnp.mean((h2 - mu) * (h2 - mu), axis=-1, keepdims=True)
    y = (h2 - mu) * jax.lax.rsqrt(var + 1e-5) * g_ref[...] + beta_ref[...]

    out_ref[0] = y
    rs_ref[0] = rs
    cs_ref[0] = cs


def kernel(adj, input_emb, seq_lens, W0, b0, W1, b1, ln_gamma, ln_beta):
    B, L, _ = adj.shape
    D = W0.shape[0]
    x0 = input_emb.reshape(B, L, D)
    b0r = b0.reshape(1, D)
    b1r = b1.reshape(1, D)
    gr = ln_gamma.reshape(1, D)
    br = ln_beta.reshape(1, D)

    out, rs, cs = pl.pallas_call(
        _gcn_fused_kernel,
        grid=(B,),
        in_specs=[
            pl.BlockSpec((1, L, L), lambda b: (b, 0, 0)),
            pl.BlockSpec((1, L, D), lambda b: (b, 0, 0)),
            pl.BlockSpec((D, D), lambda b: (0, 0)),
            pl.BlockSpec((1, D), lambda b: (0, 0)),
            pl.BlockSpec((D, D), lambda b: (0, 0)),
            pl.BlockSpec((1, D), lambda b: (0, 0)),
            pl.BlockSpec((1, D), lambda b: (0, 0)),
            pl.BlockSpec((1, D), lambda b: (0, 0)),
        ],
        out_specs=[
            pl.BlockSpec((1, L, D), lambda b: (b, 0, 0)),
            pl.BlockSpec((1, L, 1), lambda b: (b, 0, 0)),
            pl.BlockSpec((1, 1, L), lambda b: (b, 0, 0)),
        ],
        out_shape=[
            jax.ShapeDtypeStruct((B, L, D), jnp.float32),
            jax.ShapeDtypeStruct((B, L, 1), jnp.float32),
            jax.ShapeDtypeStruct((B, 1, L), jnp.float32),
        ],
    )(adj, x0, W0, b0r, W1, b1r, gr, br)

    masks = ((rs[:, :, 0] + cs[:, 0, :]) == 0.0)[..., None]
    return (out, masks)
